# Initial kernel scaffold; baseline (speedup 1.0000x reference)
import jax
import jax.numpy as jnp
from jax.experimental import pallas as pl


def kernel(x, W, b):
    raise NotImplementedError("write your pallas kernel here")


# speedup vs baseline: 1.3117x; 1.3117x over previous
import jax
import jax.numpy as jnp
from jax.experimental import pallas as pl
from jax.experimental.pallas import tpu as pltpu

_BLOCK_M = 1000


def _linear_kernel(x_ref, w_ref, b_ref, o_ref):
    x_blk = x_ref[...]
    y = jax.lax.dot_general(
        x_blk, w_ref[...],
        dimension_numbers=(((1,), (1,)), ((), ())),
        preferred_element_type=jnp.float32,
    )
    o_ref[...] = x_blk + 0.1 * (y + b_ref[...])


def kernel(x, W, b):
    m, a = x.shape
    b2d = b.reshape(1, a)
    grid = (m // _BLOCK_M,)
    return pl.pallas_call(
        _linear_kernel,
        grid=grid,
        in_specs=[
            pl.BlockSpec((_BLOCK_M, a), lambda i: (i, 0)),
            pl.BlockSpec((a, a), lambda i: (0, 0)),
            pl.BlockSpec((1, a), lambda i: (0, 0)),
        ],
        out_specs=pl.BlockSpec((_BLOCK_M, a), lambda i: (i, 0)),
        out_shape=jax.ShapeDtypeStruct((m, a), x.dtype),
        compiler_params=pltpu.CompilerParams(
            dimension_semantics=("arbitrary",),
        ),
    )(x, W, b2d)
